# baseline (speedup 1.0000x reference)
import jax
import jax.numpy as jnp
from jax import lax
from jax.experimental import pallas as pl
from jax.experimental.pallas import tpu as pltpu

_BN_EPS = 1e-5
_OUT_ROWS = 56 * 64


def _dot_tn(a, b):
    return lax.dot_general(a, b, (((1,), (1,)), ((), ())),
                           preferred_element_type=jnp.float32)


def _stats_body(x0_ref, x1_ref, x2_ref, o0_ref, o1_ref, o2_ref):
    for x_ref, o_ref in ((x0_ref, o0_ref), (x1_ref, o1_ref), (x2_ref, o2_ref)):
        x = x_ref[0]
        gram = lax.dot_general(x, x, (((0,), (0,)), ((), ())),
                               preferred_element_type=jnp.float32)
        ssum = jnp.sum(x.astype(jnp.float32), axis=0, keepdims=True)
        o_ref[0] = jnp.concatenate([gram, ssum], axis=0)


def _run_stats(xt0, xt1, xt2):
    N = xt0.shape[0]
    lanes = [xt0.shape[2], xt1.shape[2], xt2.shape[2]]
    return pl.pallas_call(
        _stats_body,
        out_shape=tuple(
            jax.ShapeDtypeStruct((N, c + 1, c), jnp.float32) for c in lanes),
        grid=(N,),
        in_specs=[
            pl.BlockSpec((1,) + xt0.shape[1:], lambda n: (n, 0, 0)),
            pl.BlockSpec((1,) + xt1.shape[1:], lambda n: (n, 0, 0)),
            pl.BlockSpec((1,) + xt2.shape[1:], lambda n: (n, 0, 0)),
        ],
        out_specs=tuple(
            pl.BlockSpec((1, c + 1, c), lambda n: (n, 0, 0)) for c in lanes),
        compiler_params=pltpu.CompilerParams(
            dimension_semantics=("parallel",)),
    )(xt0, xt1, xt2)


def _zip_rows(lo, hi):
    ul = lax.bitcast_convert_type(lo, jnp.uint16).astype(jnp.uint32)
    uh = lax.bitcast_convert_type(hi, jnp.uint16).astype(jnp.uint32)
    w = jnp.bitwise_or(ul, jnp.left_shift(uh, 16))
    return pltpu.bitcast(lax.bitcast_convert_type(w, jnp.int32), jnp.bfloat16)


def _dup_rows(b):
    return _zip_rows(b, b)


def _rep_groups(b, g, r):
    m = b.reshape(g, 1, 64, b.shape[1])
    m = jnp.broadcast_to(m, (g, r, 64, b.shape[1]))
    return m.reshape(g * r * 64, b.shape[1])


def _apply_body(x0_ref, x1_ref, x2_ref, w0_ref, w1_ref, w2_ref, sh_ref,
                o_ref):
    t0 = _dot_tn(x0_ref[0], w0_ref[...])
    t1 = _dot_tn(x1_ref[0], w1_ref[...])
    t2 = _dot_tn(x2_ref[0], w2_ref[...])

    u0 = _zip_rows(t0[:, :128].astype(jnp.bfloat16),
                   t0[:, 128:].astype(jnp.bfloat16))
    u1 = _rep_groups(_dup_rows(t1.astype(jnp.bfloat16)), 28, 2)
    u2 = _rep_groups(_dup_rows(_dup_rows(t2.astype(jnp.bfloat16))), 14, 4)

    acc = (u0.astype(jnp.float32) + u1.astype(jnp.float32)
           + u2.astype(jnp.float32) + sh_ref[...])
    o_ref[0] = acc.reshape(56, 64, acc.shape[1])[:, :56, :]


def _run_apply(xt0, xt1, xt2, ws0, ws1, ws2, shift):
    N = xt0.shape[0]
    cout = shift.shape[1]
    return pl.pallas_call(
        _apply_body,
        out_shape=jax.ShapeDtypeStruct((N, 56, 56, cout), jnp.float32),
        grid=(N,),
        in_specs=[
            pl.BlockSpec((1,) + xt0.shape[1:], lambda n: (n, 0, 0)),
            pl.BlockSpec((1,) + xt1.shape[1:], lambda n: (n, 0, 0)),
            pl.BlockSpec((1,) + xt2.shape[1:], lambda n: (n, 0, 0)),
            pl.BlockSpec(ws0.shape, lambda n: (0, 0)),
            pl.BlockSpec(ws1.shape, lambda n: (0, 0)),
            pl.BlockSpec(ws2.shape, lambda n: (0, 0)),
            pl.BlockSpec(shift.shape, lambda n: (0, 0)),
        ],
        out_specs=pl.BlockSpec((1, 56, 56, cout), lambda n: (n, 0, 0, 0)),
        compiler_params=pltpu.CompilerParams(
            dimension_semantics=("parallel",)),
    )(xt0, xt1, xt2, ws0, ws1, ws2, shift)


def _make_glue_body(cnts, ceffs, lpads):
    def _glue_body(g0_ref, g1_ref, g2_ref, w0_ref, w1_ref, w2_ref,
                   ga0_ref, ga1_ref, ga2_ref, be0_ref, be1_ref, be2_ref,
                   ws0_ref, ws1_ref, ws2_ref, sh_ref):
        cout = ga0_ref.shape[1]
        shift_total = jnp.zeros((cout, 1), jnp.float32)
        ins = ((g0_ref, w0_ref, ga0_ref, be0_ref, ws0_ref, cnts[0],
                ceffs[0], lpads[0]),
               (g1_ref, w1_ref, ga1_ref, be1_ref, ws1_ref, cnts[1],
                ceffs[1], lpads[1]),
               (g2_ref, w2_ref, ga2_ref, be2_ref, ws2_ref, cnts[2],
                ceffs[2], lpads[2]))
        for g_ref, w_ref, ga_ref, be_ref, ws_ref, cnt, c, lp in ins:
            s = jnp.sum(g_ref[...], axis=0)
            packed = w_ref is w0_ref
            if packed:
                gram = s[:c, :c] + s[c:2 * c, c:2 * c]
                srow = s[lp:lp + 1, :c] + s[lp:lp + 1, c:2 * c]
            else:
                gram = s[:c, :c]
                srow = s[lp:lp + 1, :c]
            mean_x = jnp.transpose(srow) / cnt
            cov_x = gram / cnt - mean_x * jnp.transpose(mean_x)
            w = w_ref[...]
            mean_y = jnp.dot(w, mean_x,
                             preferred_element_type=jnp.float32)
            wc = jnp.dot(w, cov_x, preferred_element_type=jnp.float32)
            var_y = jnp.maximum(
                jnp.sum(wc * w, axis=1, keepdims=True), 0.0)
            scale = jnp.transpose(ga_ref[...]) * lax.rsqrt(var_y + _BN_EPS)
            shift_total = (shift_total + jnp.transpose(be_ref[...])
                           - mean_y * scale)
            ws = scale * w
            if packed:
                z = jnp.zeros((cout, c), jnp.float32)
                ws = jnp.concatenate(
                    [jnp.concatenate([ws, z], axis=1),
                     jnp.concatenate([z, ws], axis=1)], axis=0)
            ws_ref[...] = ws.astype(ws_ref.dtype)
        sh_ref[...] = jnp.transpose(shift_total)
    return _glue_body


def _run_glue(g0x, g1x, g2x, w0, w1, w2, ga0, ga1, ga2, be0, be1, be2,
              cnts, ceffs, lpads):
    cout = w0.shape[0]
    full = lambda a: pl.BlockSpec(a.shape, lambda: tuple(0 for _ in a.shape))
    args = (g0x, g1x, g2x, w0, w1, w2,
            ga0.reshape(1, -1), ga1.reshape(1, -1), ga2.reshape(1, -1),
            be0.reshape(1, -1), be1.reshape(1, -1), be2.reshape(1, -1))
    wshapes = ((2 * cout, 128), (cout, 128), (cout, 256))
    out_sds = tuple(
        jax.ShapeDtypeStruct(ws, jnp.bfloat16) for ws in wshapes)
    return pl.pallas_call(
        _make_glue_body(cnts, ceffs, lpads),
        out_shape=out_sds + (jax.ShapeDtypeStruct((1, cout), jnp.float32),),
        in_specs=[full(a) for a in args],
        out_specs=tuple(
            pl.BlockSpec(ws, lambda: (0, 0)) for ws in wshapes)
        + (pl.BlockSpec((1, cout), lambda: (0, 0)),),
    )(*args)


def kernel(x0, x1, x2, w0, b0, g0, be0, w1, b1, g1, be1, w2, b2, g2, be2):
    N = x0.shape[0]
    bf = jnp.bfloat16
    xt0 = jnp.pad(x0.transpose(0, 2, 3, 1),
                  ((0, 0), (0, 0), (0, 8), (0, 0))).astype(bf).reshape(
                      N, 56 * 32, 128)
    xt1 = jnp.pad(x1.transpose(0, 2, 3, 1),
                  ((0, 0), (0, 0), (0, 4), (0, 0))).astype(bf).reshape(
                      N, 28 * 32, 128)
    xt2 = jnp.pad(x2.transpose(0, 2, 3, 1),
                  ((0, 0), (0, 0), (0, 2), (0, 0))).astype(bf).reshape(
                      N, 14 * 16, 256)

    g0x, g1x, g2x = _run_stats(xt0, xt1, xt2)

    ws0, ws1, ws2, shift = _run_glue(
        g0x, g1x, g2x, w0, w1, w2, g0, g1, g2, be0, be1, be2,
        cnts=(float(N * 56 * 56), float(N * 28 * 28), float(N * 14 * 14)),
        ceffs=(64, 128, 256), lpads=(128, 128, 256))

    out = _run_apply(xt0, xt1, xt2, ws0, ws1, ws2, shift)
    return out.transpose(0, 3, 1, 2)


# speedup vs baseline: 1.2321x; 1.2321x over previous
import jax
import jax.numpy as jnp
from jax import lax
from jax.experimental import pallas as pl
from jax.experimental.pallas import tpu as pltpu

_BN_EPS = 1e-5
_OUT_ROWS = 56 * 64


def _dot_tn(a, b):
    return lax.dot_general(a, b, (((1,), (1,)), ((), ())),
                           preferred_element_type=jnp.float32)


def _stats_body(x0_ref, x1_ref, x2_ref, o0_ref, o1_ref, o2_ref):
    for x_ref, o_ref in ((x0_ref, o0_ref), (x1_ref, o1_ref), (x2_ref, o2_ref)):
        x = x_ref[0]
        gram = lax.dot_general(x, x, (((0,), (0,)), ((), ())),
                               preferred_element_type=jnp.float32)
        ssum = jnp.sum(x.astype(jnp.float32), axis=0, keepdims=True)
        o_ref[0] = jnp.concatenate([gram, ssum], axis=0)


def _run_stats(xt0, xt1, xt2):
    N = xt0.shape[0]
    lanes = [xt0.shape[2], xt1.shape[2], xt2.shape[2]]
    return pl.pallas_call(
        _stats_body,
        out_shape=tuple(
            jax.ShapeDtypeStruct((N, c + 1, c), jnp.float32) for c in lanes),
        grid=(N,),
        in_specs=[
            pl.BlockSpec((1,) + xt0.shape[1:], lambda n: (n, 0, 0)),
            pl.BlockSpec((1,) + xt1.shape[1:], lambda n: (n, 0, 0)),
            pl.BlockSpec((1,) + xt2.shape[1:], lambda n: (n, 0, 0)),
        ],
        out_specs=tuple(
            pl.BlockSpec((1, c + 1, c), lambda n: (n, 0, 0)) for c in lanes),
        compiler_params=pltpu.CompilerParams(
            dimension_semantics=("parallel",)),
    )(xt0, xt1, xt2)


def _dup_rows(b):
    u = lax.bitcast_convert_type(b, jnp.uint16).astype(jnp.uint32)
    w = jnp.bitwise_or(u, jnp.left_shift(u, 16))
    return pltpu.bitcast(lax.bitcast_convert_type(w, jnp.int32), jnp.bfloat16)


def _rep_groups(b, g, r):
    m = b.reshape(g, 1, 64, b.shape[1])
    m = jnp.broadcast_to(m, (g, r, 64, b.shape[1]))
    return m.reshape(g * r * 64, b.shape[1])


def _apply_body(x0_ref, x1_ref, x2_ref, w0_ref, w1_ref, w2_ref, sh_ref,
                o_ref):
    y0 = _dot_tn(x0_ref[0], w0_ref[...])
    t1 = _dot_tn(x1_ref[0], w1_ref[...])
    t2 = _dot_tn(x2_ref[0], w2_ref[...])

    u1 = _rep_groups(_dup_rows(t1.astype(jnp.bfloat16)), 28, 2)
    u2 = _rep_groups(_dup_rows(_dup_rows(t2.astype(jnp.bfloat16))), 14, 4)

    acc = y0 + u1.astype(jnp.float32) + u2.astype(jnp.float32) + sh_ref[...]
    o_ref[0] = acc.reshape(56, 64, acc.shape[1])[:, :56, :]


def _run_apply(xt0, xt1, xt2, ws0, ws1, ws2, shift):
    N = xt0.shape[0]
    cout = ws0.shape[0]
    return pl.pallas_call(
        _apply_body,
        out_shape=jax.ShapeDtypeStruct((N, 56, 56, cout), jnp.float32),
        grid=(N,),
        in_specs=[
            pl.BlockSpec((1,) + xt0.shape[1:], lambda n: (n, 0, 0)),
            pl.BlockSpec((1,) + xt1.shape[1:], lambda n: (n, 0, 0)),
            pl.BlockSpec((1,) + xt2.shape[1:], lambda n: (n, 0, 0)),
            pl.BlockSpec(ws0.shape, lambda n: (0, 0)),
            pl.BlockSpec(ws1.shape, lambda n: (0, 0)),
            pl.BlockSpec(ws2.shape, lambda n: (0, 0)),
            pl.BlockSpec(shift.shape, lambda n: (0, 0)),
        ],
        out_specs=pl.BlockSpec((1, 56, 56, cout), lambda n: (n, 0, 0, 0)),
        compiler_params=pltpu.CompilerParams(
            dimension_semantics=("parallel",)),
    )(xt0, xt1, xt2, ws0, ws1, ws2, shift)


def _make_glue_body(cnts, ceffs, lpads):
    def _glue_body(g0_ref, g1_ref, g2_ref, w0_ref, w1_ref, w2_ref,
                   ga0_ref, ga1_ref, ga2_ref, be0_ref, be1_ref, be2_ref,
                   ws0_ref, ws1_ref, ws2_ref, sh_ref):
        cout = ga0_ref.shape[1]
        shift_total = jnp.zeros((cout, 1), jnp.float32)
        ins = ((g0_ref, w0_ref, ga0_ref, be0_ref, ws0_ref, cnts[0],
                ceffs[0], lpads[0]),
               (g1_ref, w1_ref, ga1_ref, be1_ref, ws1_ref, cnts[1],
                ceffs[1], lpads[1]),
               (g2_ref, w2_ref, ga2_ref, be2_ref, ws2_ref, cnts[2],
                ceffs[2], lpads[2]))
        for g_ref, w_ref, ga_ref, be_ref, ws_ref, cnt, c, lp in ins:
            s = jnp.sum(g_ref[...], axis=0)
            gram = s[:c, :c]
            mean_x = jnp.transpose(s[lp:lp + 1, :c]) / cnt
            cov_x = gram / cnt - mean_x * jnp.transpose(mean_x)
            w = w_ref[...]
            mean_y = jnp.dot(w, mean_x,
                             preferred_element_type=jnp.float32)
            wc = jnp.dot(w, cov_x, preferred_element_type=jnp.float32)
            var_y = jnp.maximum(
                jnp.sum(wc * w, axis=1, keepdims=True), 0.0)
            scale = jnp.transpose(ga_ref[...]) * lax.rsqrt(var_y + _BN_EPS)
            shift_total = (shift_total + jnp.transpose(be_ref[...])
                           - mean_y * scale)
            ws = scale * w
            if lp > c:
                ws = jnp.concatenate(
                    [ws, jnp.zeros((cout, lp - c), jnp.float32)], axis=1)
            ws_ref[...] = ws.astype(ws_ref.dtype)
        sh_ref[...] = jnp.transpose(shift_total)
    return _glue_body


def _run_glue(g0x, g1x, g2x, w0, w1, w2, ga0, ga1, ga2, be0, be1, be2,
              cnts, ceffs, lpads):
    cout = w0.shape[0]
    full = lambda a: pl.BlockSpec(a.shape, lambda: tuple(0 for _ in a.shape))
    args = (g0x, g1x, g2x, w0, w1, w2,
            ga0.reshape(1, -1), ga1.reshape(1, -1), ga2.reshape(1, -1),
            be0.reshape(1, -1), be1.reshape(1, -1), be2.reshape(1, -1))
    out_sds = tuple(
        jax.ShapeDtypeStruct((cout, lp), jnp.bfloat16) for lp in lpads)
    return pl.pallas_call(
        _make_glue_body(cnts, ceffs, lpads),
        out_shape=out_sds + (jax.ShapeDtypeStruct((1, cout), jnp.float32),),
        in_specs=[full(a) for a in args],
        out_specs=tuple(
            pl.BlockSpec((cout, lp), lambda: (0, 0)) for lp in lpads)
        + (pl.BlockSpec((1, cout), lambda: (0, 0)),),
    )(*args)


def kernel(x0, x1, x2, w0, b0, g0, be0, w1, b1, g1, be1, w2, b2, g2, be2):
    N = x0.shape[0]
    bf = jnp.bfloat16
    xt0 = jnp.pad(x0.transpose(0, 2, 3, 1),
                  ((0, 0), (0, 0), (0, 8), (0, 64))).astype(bf).reshape(
                      N, 56 * 64, 128)
    xt1 = jnp.pad(x1.transpose(0, 2, 3, 1),
                  ((0, 0), (0, 0), (0, 4), (0, 0))).astype(bf).reshape(
                      N, 28 * 32, 128)
    xt2 = jnp.pad(x2.transpose(0, 2, 3, 1),
                  ((0, 0), (0, 0), (0, 2), (0, 0))).astype(bf).reshape(
                      N, 14 * 16, 256)

    g0x, g1x, g2x = _run_stats(xt0, xt1, xt2)

    ws0, ws1, ws2, shift = _run_glue(
        g0x, g1x, g2x, w0, w1, w2, g0, g1, g2, be0, be1, be2,
        cnts=(float(N * 56 * 56), float(N * 28 * 28), float(N * 14 * 14)),
        ceffs=(64, 128, 256), lpads=(128, 128, 256))

    out = _run_apply(xt0, xt1, xt2, ws0, ws1, ws2, shift)
    return out.transpose(0, 3, 1, 2)


# speedup vs baseline: 1.2351x; 1.0024x over previous
import jax
import jax.numpy as jnp
from jax import lax
from jax.experimental import pallas as pl
from jax.experimental.pallas import tpu as pltpu

_BN_EPS = 1e-5


def _dot_tn(a, b):
    return lax.dot_general(a, b, (((1,), (1,)), ((), ())),
                           preferred_element_type=jnp.float32)


def _stats_body(x0_ref, x1_ref, x2_ref, o0_ref, o1_ref, o2_ref):
    for x_ref, o_ref in ((x0_ref, o0_ref), (x1_ref, o1_ref), (x2_ref, o2_ref)):
        x = x_ref[0]
        gram = lax.dot_general(x, x, (((0,), (0,)), ((), ())),
                               preferred_element_type=jnp.float32)
        ssum = jnp.sum(x.astype(jnp.float32), axis=0, keepdims=True)
        o_ref[0] = jnp.concatenate([gram, ssum], axis=0)


def _run_stats(xt0, xt1, xt2):
    N = xt0.shape[0]
    lanes = [xt0.shape[2], xt1.shape[2], xt2.shape[2]]
    return pl.pallas_call(
        _stats_body,
        out_shape=tuple(
            jax.ShapeDtypeStruct((N, c + 1, c), jnp.float32) for c in lanes),
        grid=(N,),
        in_specs=[
            pl.BlockSpec((1,) + xt0.shape[1:], lambda n: (n, 0, 0)),
            pl.BlockSpec((1,) + xt1.shape[1:], lambda n: (n, 0, 0)),
            pl.BlockSpec((1,) + xt2.shape[1:], lambda n: (n, 0, 0)),
        ],
        out_specs=tuple(
            pl.BlockSpec((1, c + 1, c), lambda n: (n, 0, 0)) for c in lanes),
        compiler_params=pltpu.CompilerParams(
            dimension_semantics=("parallel",)),
    )(xt0, xt1, xt2)


def _dup_rows(b):
    u = lax.bitcast_convert_type(b, jnp.uint16).astype(jnp.uint32)
    w = jnp.bitwise_or(u, jnp.left_shift(u, 16))
    return pltpu.bitcast(lax.bitcast_convert_type(w, jnp.int32), jnp.bfloat16)


def _rep_groups(b, g, r):
    m = b.reshape(g, 1, 64, b.shape[1])
    m = jnp.broadcast_to(m, (g, r, 64, b.shape[1]))
    return m.reshape(g * r * 64, b.shape[1])


def _apply_body(x0_ref, x1_ref, x2_ref, w0_ref, w1_ref, w2_ref, sh_ref,
                o_ref):
    y0 = _dot_tn(x0_ref[0], w0_ref[...])
    t1 = _dot_tn(x1_ref[0], w1_ref[...])
    t2 = _dot_tn(x2_ref[0], w2_ref[...])

    u1 = _rep_groups(_dup_rows(t1.astype(jnp.bfloat16)), 28, 2)
    u2 = _rep_groups(_dup_rows(_dup_rows(t2.astype(jnp.bfloat16))), 14, 4)

    acc = y0 + u1.astype(jnp.float32) + u2.astype(jnp.float32) + sh_ref[...]
    o_ref[0] = acc.reshape(56, 64, acc.shape[1])[:, :56, :]


def _run_apply(xt0, xt1, xt2, ws0, ws1, ws2, shift):
    N = xt0.shape[0]
    cout = ws0.shape[0]
    return pl.pallas_call(
        _apply_body,
        out_shape=jax.ShapeDtypeStruct((N, 56, 56, cout), jnp.float32),
        grid=(N,),
        in_specs=[
            pl.BlockSpec((1,) + xt0.shape[1:], lambda n: (n, 0, 0)),
            pl.BlockSpec((1,) + xt1.shape[1:], lambda n: (n, 0, 0)),
            pl.BlockSpec((1,) + xt2.shape[1:], lambda n: (n, 0, 0)),
            pl.BlockSpec(ws0.shape, lambda n: (0, 0)),
            pl.BlockSpec(ws1.shape, lambda n: (0, 0)),
            pl.BlockSpec(ws2.shape, lambda n: (0, 0)),
            pl.BlockSpec(shift.shape, lambda n: (0, 0)),
        ],
        out_specs=pl.BlockSpec((1, 56, 56, cout), lambda n: (n, 0, 0, 0)),
        compiler_params=pltpu.CompilerParams(
            dimension_semantics=("parallel",)),
    )(xt0, xt1, xt2, ws0, ws1, ws2, shift)


def _make_glue_body(cnts, ceffs, lpads):
    def _glue_body(g0_ref, g1_ref, g2_ref, w0_ref, w1_ref, w2_ref,
                   ga0_ref, ga1_ref, ga2_ref, be0_ref, be1_ref, be2_ref,
                   ws0_ref, ws1_ref, ws2_ref, sh_ref):
        cout = ga0_ref.shape[1]
        shift_total = jnp.zeros((cout, 1), jnp.float32)
        ins = ((g0_ref, w0_ref, ga0_ref, be0_ref, ws0_ref, cnts[0],
                ceffs[0], lpads[0]),
               (g1_ref, w1_ref, ga1_ref, be1_ref, ws1_ref, cnts[1],
                ceffs[1], lpads[1]),
               (g2_ref, w2_ref, ga2_ref, be2_ref, ws2_ref, cnts[2],
                ceffs[2], lpads[2]))
        for g_ref, w_ref, ga_ref, be_ref, ws_ref, cnt, c, lp in ins:
            s = jnp.sum(g_ref[...], axis=0)
            gram = s[:c, :c]
            mean_x = jnp.transpose(s[lp:lp + 1, :c]) / cnt
            cov_x = gram / cnt - mean_x * jnp.transpose(mean_x)
            w = w_ref[...]
            mean_y = jnp.dot(w, mean_x,
                             preferred_element_type=jnp.float32)
            wc = jnp.dot(w, cov_x, preferred_element_type=jnp.float32)
            var_y = jnp.maximum(
                jnp.sum(wc * w, axis=1, keepdims=True), 0.0)
            scale = jnp.transpose(ga_ref[...]) * lax.rsqrt(var_y + _BN_EPS)
            shift_total = (shift_total + jnp.transpose(be_ref[...])
                           - mean_y * scale)
            ws = scale * w
            if lp > c:
                ws = jnp.concatenate(
                    [ws, jnp.zeros((cout, lp - c), jnp.float32)], axis=1)
            ws_ref[...] = ws.astype(ws_ref.dtype)
        sh_ref[...] = jnp.transpose(shift_total)
    return _glue_body


def _run_glue(g0x, g1x, g2x, w0, w1, w2, ga0, ga1, ga2, be0, be1, be2,
              cnts, ceffs, lpads):
    cout = w0.shape[0]
    full = lambda a: pl.BlockSpec(a.shape, lambda: tuple(0 for _ in a.shape))
    args = (g0x, g1x, g2x, w0, w1, w2,
            ga0.reshape(1, -1), ga1.reshape(1, -1), ga2.reshape(1, -1),
            be0.reshape(1, -1), be1.reshape(1, -1), be2.reshape(1, -1))
    out_sds = tuple(
        jax.ShapeDtypeStruct((cout, lp), jnp.bfloat16) for lp in lpads)
    return pl.pallas_call(
        _make_glue_body(cnts, ceffs, lpads),
        out_shape=out_sds + (jax.ShapeDtypeStruct((1, cout), jnp.float32),),
        in_specs=[full(a) for a in args],
        out_specs=tuple(
            pl.BlockSpec((cout, lp), lambda: (0, 0)) for lp in lpads)
        + (pl.BlockSpec((1, cout), lambda: (0, 0)),),
    )(*args)


def kernel(x0, x1, x2, w0, b0, g0, be0, w1, b1, g1, be1, w2, b2, g2, be2):
    N = x0.shape[0]
    bf = jnp.bfloat16
    xt0 = jnp.pad(x0.transpose(0, 2, 3, 1),
                  ((0, 0), (0, 0), (0, 8), (0, 64))).astype(bf).reshape(
                      N, 56 * 64, 128)
    xt1 = jnp.pad(x1.transpose(0, 2, 3, 1),
                  ((0, 0), (0, 0), (0, 4), (0, 0))).astype(bf).reshape(
                      N, 28 * 32, 128)
    xt2 = jnp.pad(x2.transpose(0, 2, 3, 1),
                  ((0, 0), (0, 0), (0, 2), (0, 0))).astype(bf).reshape(
                      N, 14 * 16, 256)

    g0x, g1x, g2x = _run_stats(xt0, xt1, xt2)

    ws0, ws1, ws2, shift = _run_glue(
        g0x, g1x, g2x, w0, w1, w2, g0, g1, g2, be0, be1, be2,
        cnts=(float(N * 56 * 56), float(N * 28 * 28), float(N * 14 * 14)),
        ceffs=(64, 128, 256), lpads=(128, 128, 256))

    out = _run_apply(xt0, xt1, xt2, ws0, ws1, ws2, shift)
    return out.transpose(0, 3, 1, 2)


# speedup vs baseline: 1.2920x; 1.0461x over previous
import jax
import jax.numpy as jnp
from jax import lax
from jax.experimental import pallas as pl
from jax.experimental.pallas import tpu as pltpu

_BN_EPS = 1e-5


def _dot_tn(a, b):
    return lax.dot_general(a, b, (((1,), (1,)), ((), ())),
                           preferred_element_type=jnp.float32)


def _stats_body(x0_ref, x1_ref, x2_ref, o0_ref, o1_ref, o2_ref):
    for x_ref, o_ref in ((x0_ref, o0_ref), (x1_ref, o1_ref), (x2_ref, o2_ref)):
        x = x_ref[0]
        gram = lax.dot_general(x, x, (((0,), (0,)), ((), ())),
                               preferred_element_type=jnp.float32)
        ssum = jnp.sum(x.astype(jnp.float32), axis=0, keepdims=True)
        o_ref[0] = jnp.concatenate([gram, ssum], axis=0)


def _run_stats(xt0, xt1, xt2):
    N = xt0.shape[0]
    lanes = [xt0.shape[2], xt1.shape[2], xt2.shape[2]]
    return pl.pallas_call(
        _stats_body,
        out_shape=tuple(
            jax.ShapeDtypeStruct((N, c + 1, c), jnp.float32) for c in lanes),
        grid=(N,),
        in_specs=[
            pl.BlockSpec((1,) + xt0.shape[1:], lambda n: (n, 0, 0)),
            pl.BlockSpec((1,) + xt1.shape[1:], lambda n: (n, 0, 0)),
            pl.BlockSpec((1,) + xt2.shape[1:], lambda n: (n, 0, 0)),
        ],
        out_specs=tuple(
            pl.BlockSpec((1, c + 1, c), lambda n: (n, 0, 0)) for c in lanes),
        compiler_params=pltpu.CompilerParams(
            dimension_semantics=("parallel",)),
    )(xt0, xt1, xt2)


def _dup_rows(b):
    u = lax.bitcast_convert_type(b, jnp.uint16).astype(jnp.uint32)
    w = jnp.bitwise_or(u, jnp.left_shift(u, 16))
    return pltpu.bitcast(lax.bitcast_convert_type(w, jnp.int32), jnp.bfloat16)


def _rep_groups(b, g, r):
    s = b.shape[0] // g
    m = b.reshape(g, 1, s, b.shape[1])
    m = jnp.broadcast_to(m, (g, r, s, b.shape[1]))
    return m.reshape(g * r * s, b.shape[1])


def _apply_body(x0_ref, x1_ref, x2_ref, w0_ref, w1_ref, w2_ref, sh_ref,
                o_ref):
    y0 = _dot_tn(x0_ref[0], w0_ref[...])
    t1 = _dot_tn(x1_ref[0], w1_ref[...])
    t2 = _dot_tn(x2_ref[0], w2_ref[...])

    u1 = _rep_groups(_dup_rows(t1.astype(jnp.bfloat16)), 28, 2)
    u2 = _rep_groups(_dup_rows(_dup_rows(t2.astype(jnp.bfloat16))), 14, 4)

    acc = y0 + u1.astype(jnp.float32) + u2.astype(jnp.float32) + sh_ref[...]
    o_ref[0] = acc.reshape(56, 56, acc.shape[1])


def _run_apply(xt0, xt1, xt2, ws0, ws1, ws2, shift):
    N = xt0.shape[0]
    cout = ws0.shape[0]
    return pl.pallas_call(
        _apply_body,
        out_shape=jax.ShapeDtypeStruct((N, 56, 56, cout), jnp.float32),
        grid=(N,),
        in_specs=[
            pl.BlockSpec((1,) + xt0.shape[1:], lambda n: (n, 0, 0)),
            pl.BlockSpec((1,) + xt1.shape[1:], lambda n: (n, 0, 0)),
            pl.BlockSpec((1,) + xt2.shape[1:], lambda n: (n, 0, 0)),
            pl.BlockSpec(ws0.shape, lambda n: (0, 0)),
            pl.BlockSpec(ws1.shape, lambda n: (0, 0)),
            pl.BlockSpec(ws2.shape, lambda n: (0, 0)),
            pl.BlockSpec(shift.shape, lambda n: (0, 0)),
        ],
        out_specs=pl.BlockSpec((1, 56, 56, cout), lambda n: (n, 0, 0, 0)),
        compiler_params=pltpu.CompilerParams(
            dimension_semantics=("parallel",)),
    )(xt0, xt1, xt2, ws0, ws1, ws2, shift)


def _make_glue_body(cnts, ceffs, lpads):
    def _glue_body(g0_ref, g1_ref, g2_ref, w0_ref, w1_ref, w2_ref,
                   ga0_ref, ga1_ref, ga2_ref, be0_ref, be1_ref, be2_ref,
                   ws0_ref, ws1_ref, ws2_ref, sh_ref):
        cout = ga0_ref.shape[1]
        shift_total = jnp.zeros((cout, 1), jnp.float32)
        ins = ((g0_ref, w0_ref, ga0_ref, be0_ref, ws0_ref, cnts[0],
                ceffs[0], lpads[0]),
               (g1_ref, w1_ref, ga1_ref, be1_ref, ws1_ref, cnts[1],
                ceffs[1], lpads[1]),
               (g2_ref, w2_ref, ga2_ref, be2_ref, ws2_ref, cnts[2],
                ceffs[2], lpads[2]))
        for g_ref, w_ref, ga_ref, be_ref, ws_ref, cnt, c, lp in ins:
            s = jnp.sum(g_ref[...], axis=0)
            gram = s[:c, :c]
            mean_x = jnp.transpose(s[lp:lp + 1, :c]) / cnt
            cov_x = gram / cnt - mean_x * jnp.transpose(mean_x)
            w = w_ref[...]
            mean_y = jnp.dot(w, mean_x,
                             preferred_element_type=jnp.float32)
            wc = jnp.dot(w, cov_x, preferred_element_type=jnp.float32)
            var_y = jnp.maximum(
                jnp.sum(wc * w, axis=1, keepdims=True), 0.0)
            scale = jnp.transpose(ga_ref[...]) * lax.rsqrt(var_y + _BN_EPS)
            shift_total = (shift_total + jnp.transpose(be_ref[...])
                           - mean_y * scale)
            ws = scale * w
            if lp > c:
                ws = jnp.concatenate(
                    [ws, jnp.zeros((cout, lp - c), jnp.float32)], axis=1)
            ws_ref[...] = ws.astype(ws_ref.dtype)
        sh_ref[...] = jnp.transpose(shift_total)
    return _glue_body


def _run_glue(g0x, g1x, g2x, w0, w1, w2, ga0, ga1, ga2, be0, be1, be2,
              cnts, ceffs, lpads):
    cout = w0.shape[0]
    full = lambda a: pl.BlockSpec(a.shape, lambda: tuple(0 for _ in a.shape))
    args = (g0x, g1x, g2x, w0, w1, w2,
            ga0.reshape(1, -1), ga1.reshape(1, -1), ga2.reshape(1, -1),
            be0.reshape(1, -1), be1.reshape(1, -1), be2.reshape(1, -1))
    out_sds = tuple(
        jax.ShapeDtypeStruct((cout, lp), jnp.bfloat16) for lp in lpads)
    return pl.pallas_call(
        _make_glue_body(cnts, ceffs, lpads),
        out_shape=out_sds + (jax.ShapeDtypeStruct((1, cout), jnp.float32),),
        in_specs=[full(a) for a in args],
        out_specs=tuple(
            pl.BlockSpec((cout, lp), lambda: (0, 0)) for lp in lpads)
        + (pl.BlockSpec((1, cout), lambda: (0, 0)),),
    )(*args)


def kernel(x0, x1, x2, w0, b0, g0, be0, w1, b1, g1, be1, w2, b2, g2, be2):
    N = x0.shape[0]
    bf = jnp.bfloat16
    xt0 = jnp.pad(x0.transpose(0, 2, 3, 1),
                  ((0, 0), (0, 0), (0, 0), (0, 64))).astype(bf).reshape(
                      N, 56 * 56, 128)
    xt1 = x1.transpose(0, 2, 3, 1).astype(bf).reshape(N, 28 * 28, 128)
    xt2 = x2.transpose(0, 2, 3, 1).astype(bf).reshape(N, 14 * 14, 256)

    g0x, g1x, g2x = _run_stats(xt0, xt1, xt2)

    ws0, ws1, ws2, shift = _run_glue(
        g0x, g1x, g2x, w0, w1, w2, g0, g1, g2, be0, be1, be2,
        cnts=(float(N * 56 * 56), float(N * 28 * 28), float(N * 14 * 14)),
        ceffs=(64, 128, 256), lpads=(128, 128, 256))

    out = _run_apply(xt0, xt1, xt2, ws0, ws1, ws2, shift)
    return out.transpose(0, 3, 1, 2)
